# baseline (speedup 1.0000x reference)
import functools

import jax
import jax.numpy as jnp
from jax.experimental import pallas as pl
from jax.experimental.pallas import tpu as pltpu


def _body(n, m, chunk, y_ref, yt_ref, ref_ref, out_ref):
    nchunks = n // chunk
    y = y_ref[...]
    a_col = y[:, 0:1]
    b_col = y[:, 1:2]
    yt = yt_ref[...]
    a_row = yt[0:1, :]
    b_row = yt[1:2, :]
    ref0 = ref_ref[0]
    ref1 = ref_ref[1]
    ref_row = jnp.concatenate(
        [jnp.full((n, 1), ref0, jnp.float32), jnp.full((n, 1), ref1, jnp.float32)],
        axis=1,
    )
    inf = jnp.float32(jnp.inf)

    key_row_chunks = []
    for c in range(nchunks):
        ar = jax.lax.slice(a_row, (0, c * chunk), (1, (c + 1) * chunk))
        br = jax.lax.slice(b_row, (0, c * chunk), (1, (c + 1) * chunk))
        ge = (a_col >= ar) & (b_col >= br)
        gt = (a_col > ar) | (b_col > br)
        dom = jnp.any(ge & gt, axis=0, keepdims=True)
        feas = (ar > ref0) & (br > ref1) & (~dom)
        key_row_chunks.append(jnp.where(feas, -ar, inf))
    key_row = jnp.concatenate(key_row_chunks, axis=1)

    key_col_chunks = []
    padded_chunks = []
    for c in range(nchunks):
        ac = jax.lax.slice(a_col, (c * chunk, 0), ((c + 1) * chunk, 1))
        bc = jax.lax.slice(b_col, (c * chunk, 0), ((c + 1) * chunk, 1))
        ge = (a_row >= ac) & (b_row >= bc)
        gt = (a_row > ac) | (b_row > bc)
        dom = jnp.any(ge & gt, axis=1, keepdims=True)
        feas = (ac > ref0) & (bc > ref1) & (~dom)
        key_col_chunks.append(jnp.where(feas, -ac, inf))
        yc = jax.lax.slice(y, (c * chunk, 0), ((c + 1) * chunk, m))
        rc = jax.lax.slice(ref_row, (c * chunk, 0), ((c + 1) * chunk, m))
        padded_chunks.append(jnp.where(feas, yc, rc))
    key_col = jnp.concatenate(key_col_chunks, axis=0)
    padded = jnp.concatenate(padded_chunks, axis=0)

    j_col = jax.lax.broadcasted_iota(jnp.int32, (n, 1), 0)
    rank_chunks = []
    for c in range(nchunks):
        ki = jax.lax.slice(key_row, (0, c * chunk), (1, (c + 1) * chunk))
        i_idx = jax.lax.broadcasted_iota(jnp.int32, (1, chunk), 1) + c * chunk
        pred = (key_col < ki) | ((key_col == ki) & (j_col < i_idx))
        rank_chunks.append(jnp.sum(pred.astype(jnp.int32), axis=0, keepdims=True))
    rank_row = jnp.concatenate(rank_chunks, axis=1)

    for c in range(nchunks):
        p_vals = jax.lax.broadcasted_iota(jnp.int32, (chunk, 1), 0) + c * chunk
        onehot = (rank_row == p_vals).astype(jnp.float32)
        out_ref[c * chunk:(c + 1) * chunk, :] = jax.lax.dot_general(
            onehot, padded, (((1,), (0,)), ((), ())),
            precision=jax.lax.Precision.HIGHEST,
            preferred_element_type=jnp.float32)


def kernel(Y, ref_point):
    n, m = Y.shape
    chunk = 512
    body = functools.partial(_body, n, m, chunk)
    return pl.pallas_call(
        body,
        out_shape=jax.ShapeDtypeStruct((n, m), jnp.float32),
        in_specs=[
            pl.BlockSpec(memory_space=pltpu.VMEM),
            pl.BlockSpec(memory_space=pltpu.VMEM),
            pl.BlockSpec(memory_space=pltpu.SMEM),
        ],
        out_specs=pl.BlockSpec(memory_space=pltpu.VMEM),
    )(Y, Y.T, ref_point)


# speedup vs baseline: 14.9707x; 14.9707x over previous
import functools

import jax
import jax.numpy as jnp
from jax.experimental import pallas as pl
from jax.experimental.pallas import tpu as pltpu


def _body(n, rows, cols, a_ref, b_ref, ref_ref, out_ref):
    a = a_ref[...]
    b = b_ref[...]
    ref0 = ref_ref[0]
    ref1 = ref_ref[1]
    neg_inf = jnp.float32(-jnp.inf)

    col_sel = jax.lax.broadcasted_iota(jnp.int32, (n, 2), 1)
    out_ref[...] = jnp.where(col_sel == 0, ref0, ref1)

    flat_idx = (jax.lax.broadcasted_iota(jnp.int32, (rows, cols), 0) * cols
                + jax.lax.broadcasted_iota(jnp.int32, (rows, cols), 1))

    active0 = ((a > ref0) & (b > ref1)).astype(jnp.float32)

    def cond(carry):
        _, active = carry
        return jnp.max(active) > 0.0

    def body(carry):
        t, active = carry
        act = active > 0.0
        m_a = jnp.max(jnp.where(act, a, neg_inf))
        m_b = jnp.max(jnp.where(act & (a == m_a), b, neg_inf))
        out_ref[pl.ds(t, 1), :] = jnp.concatenate(
            [jnp.full((1, 1), m_a, jnp.float32),
             jnp.full((1, 1), m_b, jnp.float32)], axis=1)
        eq = act & (a == m_a) & (b == m_b)
        j0 = jnp.min(jnp.where(eq, flat_idx, n))
        strictly_dominated = (((a <= m_a) & (b < m_b)) |
                              ((a < m_a) & (b <= m_b)))
        keep = act & (~strictly_dominated) & (flat_idx != j0)
        return t + 1, keep.astype(jnp.float32)

    jax.lax.while_loop(cond, body, (jnp.int32(0), active0))


def kernel(Y, ref_point):
    n, m = Y.shape
    rows, cols = n // 128, 128
    body = functools.partial(_body, n, rows, cols)
    a2 = Y[:, 0].reshape(rows, cols)
    b2 = Y[:, 1].reshape(rows, cols)
    return pl.pallas_call(
        body,
        out_shape=jax.ShapeDtypeStruct((n, m), jnp.float32),
        in_specs=[
            pl.BlockSpec(memory_space=pltpu.VMEM),
            pl.BlockSpec(memory_space=pltpu.VMEM),
            pl.BlockSpec(memory_space=pltpu.SMEM),
        ],
        out_specs=pl.BlockSpec(memory_space=pltpu.VMEM),
    )(a2, b2, ref_point)
